# baseline (speedup 1.0000x reference)
import functools

import jax
import jax.numpy as jnp
import numpy as np
from jax.experimental import pallas as pl
from jax.experimental.pallas import tpu as pltpu


def _path_graph_dinv(num_joints: int):
    deg = np.full(num_joints, 3.0, dtype=np.float32)
    deg[0] = deg[-1] = 2.0
    return 1.0 / np.sqrt(deg)


def _decoder_body(z_ref, dv_ref, dv2_ref, dvi_ref,
                  Wg0_ref, bg0_ref, Wt0_ref, bt0_ref,
                  Wg1_ref, bg1_ref, Wt1_ref, bt1_ref,
                  Wg2_ref, bg2_ref, Wt2_ref, bt2_ref,
                  Wout_ref, bout_ref, out_ref, *, num_joints: int,
                  num_t: int):
    T, J, d = num_t, num_joints, z_ref.shape[-1]
    rows = T * J
    dv = dv_ref[...]
    dv2 = dv2_ref[...].astype(jnp.bfloat16)
    dvi = dvi_ref[...]

    r = (dv * z_ref[0].reshape(T, J, d)).astype(jnp.bfloat16)

    blocks = ((Wg0_ref, bg0_ref, Wt0_ref, bt0_ref),
              (Wg1_ref, bg1_ref, Wt1_ref, bt1_ref),
              (Wg2_ref, bg2_ref, Wt2_ref, bt2_ref))
    for Wg_ref, bg_ref, Wt_ref, bt_ref in blocks:
        zslab = jnp.zeros((T, 1, d), jnp.bfloat16)
        v = (r
             + jnp.concatenate([zslab, r[:, :-1, :]], axis=1)
             + jnp.concatenate([r[:, 1:, :], zslab], axis=1)) * dv2
        mv = jnp.dot(v.reshape(rows, d), Wg_ref[...],
                     preferred_element_type=jnp.float32).reshape(T, J, d)
        r = (r.astype(jnp.float32)
             + jnp.maximum(mv + bg_ref[...], 0.0)).astype(jnp.bfloat16)

        rf = r.reshape(rows, d)
        ztile = jnp.zeros((J, d), jnp.bfloat16)
        r_dn = jnp.concatenate([ztile, rf[:-J, :]], axis=0)
        r_up = jnp.concatenate([rf[J:, :], ztile], axis=0)
        y = (jnp.dot(r_dn, Wt_ref[0], preferred_element_type=jnp.float32)
             + jnp.dot(rf, Wt_ref[1], preferred_element_type=jnp.float32)
             + jnp.dot(r_up, Wt_ref[2], preferred_element_type=jnp.float32)
             ).reshape(T, J, d)
        r = (r.astype(jnp.float32)
             + jnp.maximum(y + bt_ref[...], 0.0)).astype(jnp.bfloat16)

    h = (r.astype(jnp.float32) * dvi).reshape(rows, d)
    out_ref[0] = (jnp.dot(h, Wout_ref[...], preferred_element_type=jnp.float32)
                  + bout_ref[...])


def kernel(z, Wg0, bg0, Wt0, bt0, Wg1, bg1, Wt1, bt1, Wg2, bg2, Wt2, bt2,
           Wout, bout):
    B, T, J, D = z.shape
    TJ = T * J
    zr = z.reshape(B, TJ, D)

    dvj = _path_graph_dinv(J)
    dv = jnp.asarray(dvj)[None, :, None]
    dv2 = jnp.asarray(dvj * dvj)[None, :, None]
    dvi = jnp.asarray(1.0 / dvj)[None, :, None]

    dcol = jnp.asarray(dvj)[:, None]
    dbg = [(dcol * b[None, :])[None] for b in (bg0, bg1, bg2)]
    dbt = [(dcol * b[None, :])[None] for b in (bt0, bt1, bt2)]

    Wts = [jnp.transpose(W, (2, 1, 0)).astype(jnp.bfloat16)
           for W in (Wt0, Wt1, Wt2)]
    Wgs = [W.astype(jnp.bfloat16) for W in (Wg0, Wg1, Wg2)]

    full = lambda shape: pl.BlockSpec(shape, lambda b: (0,) * len(shape))
    wspecs = []
    for _ in range(3):
        wspecs += [full((D, D)), full((1, J, D)), full((3, D, D)),
                   full((1, J, D))]

    out = pl.pallas_call(
        functools.partial(_decoder_body, num_joints=J, num_t=T),
        grid=(B,),
        in_specs=[pl.BlockSpec((1, TJ, D), lambda b: (b, 0, 0)),
                  full((1, J, 1)), full((1, J, 1)), full((1, J, 1)),
                  *wspecs,
                  full((D, 3)), full((1, 3))],
        out_specs=pl.BlockSpec((1, TJ, 3), lambda b: (b, 0, 0)),
        out_shape=jax.ShapeDtypeStruct((B, TJ, 3), jnp.float32),
        compiler_params=pltpu.CompilerParams(
            dimension_semantics=("parallel",)),
    )(zr, dv, dv2, dvi,
      Wgs[0], dbg[0], Wts[0], dbt[0],
      Wgs[1], dbg[1], Wts[1], dbt[1],
      Wgs[2], dbg[2], Wts[2], dbt[2],
      Wout, bout.reshape(1, 3))
    return out.reshape(B, T, J, 3)
